# baseline (speedup 1.0000x reference)
import functools

import jax
import jax.numpy as jnp
from jax import lax
from jax.experimental import pallas as pl
from jax.experimental.pallas import tpu as pltpu
from jax.experimental.pallas import tpu_sc as plsc

N = 10000
E = 320000
D = 128
DOUT = 64
NW = 32
CHUNK = 128
CH0 = 58
CH1 = 100
CHMAX = max(CH0, CH1)
EP = 16 * (CH0 + CH1) * CHUNK
NP_ROWS = 10112
RPT = NP_ROWS // 16
BN = 1000


def _sc_mesh():
    return plsc.VectorSubcoreMesh(core_axis_name="c", subcore_axis_name="s")


def _sc_degree(dstp, ones128, zerosD):
    @functools.partial(
        pl.kernel,
        mesh=_sc_mesh(),
        out_type=jax.ShapeDtypeStruct((2, NP_ROWS, D), jnp.float32),
        scratch_types=[
            pltpu.VMEM_SHARED((NP_ROWS, D), jnp.float32),
            pltpu.VMEM((CHMAX, CHUNK), jnp.int32),
            pltpu.VMEM((CHUNK, D), jnp.float32),
        ],
    )
    def k(dst0_hbm, dst1_hbm, ones_hbm, zeros_hbm, out_hbm, acc, dst_v, ones_v):
        c = lax.axis_index("c")
        s = lax.axis_index("s")
        pltpu.sync_copy(ones_hbm, ones_v)
        pltpu.sync_copy(zeros_hbm.at[pl.ds(s * RPT, RPT)],
                        acc.at[pl.ds(s * RPT, RPT)])
        plsc.subcore_barrier()

        def body(j, carry):
            pltpu.sync_copy(ones_v, acc.at[dst_v.at[j]], add=True)
            return carry

        @pl.when(c == 0)
        def _():
            pltpu.sync_copy(dst0_hbm.at[s], dst_v.at[pl.ds(0, CH0)])
            lax.fori_loop(0, CH0, body, 0)

        @pl.when(c == 1)
        def _():
            pltpu.sync_copy(dst1_hbm.at[s], dst_v.at[pl.ds(0, CH1)])
            lax.fori_loop(0, CH1, body, 0)

        plsc.subcore_barrier()
        pltpu.sync_copy(acc.at[pl.ds(s * RPT, RPT)],
                        out_hbm.at[c, pl.ds(s * RPT, RPT)])

    return k(dstp[0], dstp[1], ones128, zerosD)


def _sc_scatter(g, srcp, dstp, zerosD):
    @functools.partial(
        pl.kernel,
        mesh=_sc_mesh(),
        out_type=jax.ShapeDtypeStruct((2, NP_ROWS, D), jnp.float32),
        scratch_types=[
            pltpu.VMEM_SHARED((NP_ROWS, D), jnp.float32),
            pltpu.VMEM((CHMAX, CHUNK), jnp.int32),
            pltpu.VMEM((CHMAX, CHUNK), jnp.int32),
            pltpu.VMEM((CHUNK, D), jnp.float32),
            pltpu.SemaphoreType.DMA,
        ],
    )
    def k(g_hbm, src0_hbm, dst0_hbm, src1_hbm, dst1_hbm, zeros_hbm, out_hbm,
          acc, src_v, dst_v, rows_v, sem):
        c = lax.axis_index("c")
        s = lax.axis_index("s")
        pltpu.sync_copy(zeros_hbm.at[pl.ds(s * RPT, RPT)],
                        acc.at[pl.ds(s * RPT, RPT)])
        plsc.subcore_barrier()

        def body(j, carry):
            pltpu.async_copy(g_hbm.at[src_v.at[j]], rows_v, sem).wait()
            pltpu.sync_copy(rows_v, acc.at[dst_v.at[j]], add=True)
            return carry

        @pl.when(c == 0)
        def _():
            pltpu.sync_copy(src0_hbm.at[s], src_v.at[pl.ds(0, CH0)])
            pltpu.sync_copy(dst0_hbm.at[s], dst_v.at[pl.ds(0, CH0)])
            lax.fori_loop(0, CH0, body, 0)

        @pl.when(c == 1)
        def _():
            pltpu.sync_copy(src1_hbm.at[s], src_v.at[pl.ds(0, CH1)])
            pltpu.sync_copy(dst1_hbm.at[s], dst_v.at[pl.ds(0, CH1)])
            lax.fori_loop(0, CH1, body, 0)

        plsc.subcore_barrier()
        pltpu.sync_copy(acc.at[pl.ds(s * RPT, RPT)],
                        out_hbm.at[c, pl.ds(s * RPT, RPT)])

    return k(g, srcp[0], dstp[0], srcp[1], dstp[1], zerosD)


def _tc_pre(x, W0, i0, i1):
    def body(x_ref, w_ref, i0_ref, i1_ref, dinv_ref, g_ref):
        deg = 1.0 + i0_ref[:, :1] + i1_ref[:, :1]
        dinv = jnp.broadcast_to(lax.rsqrt(deg), (BN, D))
        dinv_ref[...] = dinv
        g_ref[...] = dinv * jnp.dot(x_ref[...], w_ref[...],
                                    preferred_element_type=jnp.float32)

    return pl.pallas_call(
        body,
        grid=(N // BN,),
        in_specs=[
            pl.BlockSpec((BN, D), lambda i: (i, 0)),
            pl.BlockSpec((D, D), lambda i: (0, 0)),
            pl.BlockSpec((BN, D), lambda i: (i, 0)),
            pl.BlockSpec((BN, D), lambda i: (i, 0)),
        ],
        out_specs=[pl.BlockSpec((BN, D), lambda i: (i, 0))] * 2,
        out_shape=[jax.ShapeDtypeStruct((N, D), jnp.float32)] * 2,
    )(x, W0, i0, i1)


def _tc_layer(S, g, dinv, b, Wn):
    def body(s0_ref, s1_ref, g_ref, dinv_ref, b_ref, w_ref, h_ref, gn_ref):
        dinv = dinv_ref[...]
        agg = dinv * (s0_ref[...] + s1_ref[...] + g_ref[...]) + b_ref[...]
        h = jnp.maximum(agg, 0.0)
        h_ref[...] = h
        gn_ref[...] = dinv * jnp.dot(h, w_ref[...],
                                     preferred_element_type=jnp.float32)

    return pl.pallas_call(
        body,
        grid=(N // BN,),
        in_specs=[
            pl.BlockSpec((BN, D), lambda i: (i, 0)),
            pl.BlockSpec((BN, D), lambda i: (i, 0)),
            pl.BlockSpec((BN, D), lambda i: (i, 0)),
            pl.BlockSpec((BN, D), lambda i: (i, 0)),
            pl.BlockSpec((1, D), lambda i: (0, 0)),
            pl.BlockSpec((D, D), lambda i: (0, 0)),
        ],
        out_specs=[pl.BlockSpec((BN, D), lambda i: (i, 0))] * 2,
        out_shape=[jax.ShapeDtypeStruct((N, D), jnp.float32)] * 2,
    )(S[0, :N, :], S[1, :N, :], g, dinv, b, Wn)


def _tc_final(S, g, dinv, b, Wffn):
    def body(s0_ref, s1_ref, g_ref, dinv_ref, b_ref, w_ref,
             h_ref, f_ref, ls_ref, sm_ref):
        dinv = dinv_ref[...]
        agg = dinv * (s0_ref[...] + s1_ref[...] + g_ref[...]) + b_ref[...]
        h = jnp.maximum(agg, 0.0)
        h_ref[...] = h
        f = jnp.maximum(jnp.dot(h, w_ref[...],
                                preferred_element_type=jnp.float32), 0.0)
        m = jnp.max(f, axis=1, keepdims=True)
        lse = m + jnp.log(jnp.sum(jnp.exp(f - m), axis=1, keepdims=True))
        ls = f - lse
        f_ref[...] = f
        ls_ref[...] = ls
        sm_ref[...] = jnp.exp(ls)

    return pl.pallas_call(
        body,
        grid=(N // BN,),
        in_specs=[
            pl.BlockSpec((BN, D), lambda i: (i, 0)),
            pl.BlockSpec((BN, D), lambda i: (i, 0)),
            pl.BlockSpec((BN, D), lambda i: (i, 0)),
            pl.BlockSpec((BN, D), lambda i: (i, 0)),
            pl.BlockSpec((1, D), lambda i: (0, 0)),
            pl.BlockSpec((D, DOUT), lambda i: (0, 0)),
        ],
        out_specs=[pl.BlockSpec((BN, D), lambda i: (i, 0))]
        + [pl.BlockSpec((BN, DOUT), lambda i: (i, 0))] * 3,
        out_shape=[jax.ShapeDtypeStruct((N, D), jnp.float32)]
        + [jax.ShapeDtypeStruct((N, DOUT), jnp.float32)] * 3,
    )(S[0, :N, :], S[1, :N, :], g, dinv, b, Wffn)


def kernel(x, edge_index, batch, W0, b0, W1, b1, W2, b2, Wffn):
    src = edge_index[0]
    dst = edge_index[1]
    pad = EP - E
    E0 = 16 * CH0 * CHUNK
    src_flat = jnp.concatenate([src, jnp.zeros((pad,), jnp.int32)])
    dst_flat = jnp.concatenate([dst, jnp.full((pad,), N, jnp.int32)])
    srcp = (src_flat[:E0].reshape(16, CH0, CHUNK),
            src_flat[E0:].reshape(16, CH1, CHUNK))
    dstp = (dst_flat[:E0].reshape(16, CH0, CHUNK),
            dst_flat[E0:].reshape(16, CH1, CHUNK))
    ones128 = jnp.ones((CHUNK, D), jnp.float32)
    zerosD = jnp.zeros((NP_ROWS, D), jnp.float32)

    indeg = _sc_degree(dstp, ones128, zerosD)
    i0 = indeg[0, :N, :]
    i1 = indeg[1, :N, :]
    dinv, g0 = _tc_pre(x, W0, i0, i1)

    S0 = _sc_scatter(g0, srcp, dstp, zerosD)
    h1, g1 = _tc_layer(S0, g0, dinv, b0.reshape(1, D), W1)

    S1 = _sc_scatter(g1, srcp, dstp, zerosD)
    h2, g2 = _tc_layer(S1, g1, dinv, b1.reshape(1, D), W2)

    S2 = _sc_scatter(g2, srcp, dstp, zerosD)
    h3, ffn_out, log_soft, soft = _tc_final(S2, g2, dinv, b2.reshape(1, D), Wffn)

    return (h1, h2, h3, h3, ffn_out, log_soft, soft)


# speedup vs baseline: 1.1930x; 1.1930x over previous
import functools

import jax
import jax.numpy as jnp
from jax import lax
from jax.experimental import pallas as pl
from jax.experimental.pallas import tpu as pltpu
from jax.experimental.pallas import tpu_sc as plsc

N = 10000
E = 320000
D = 128
DOUT = 64
NW = 32
CHUNK = 128
CH0 = 100
CH1 = 58
CHMAX = max(CH0, CH1)
EP = 16 * (CH0 + CH1) * CHUNK
NP_ROWS = 10112
RPT = NP_ROWS // 16
BN = 1000


def _sc_mesh():
    return plsc.VectorSubcoreMesh(core_axis_name="c", subcore_axis_name="s")


def _sc_degree(dstp, ones128, zerosD):
    @functools.partial(
        pl.kernel,
        mesh=_sc_mesh(),
        out_type=jax.ShapeDtypeStruct((2, NP_ROWS, D), jnp.float32),
        scratch_types=[
            pltpu.VMEM_SHARED((NP_ROWS, D), jnp.float32),
            pltpu.VMEM((CHMAX, CHUNK), jnp.int32),
            pltpu.VMEM((CHUNK, D), jnp.float32),
        ],
    )
    def k(dst0_hbm, dst1_hbm, ones_hbm, zeros_hbm, out_hbm, acc, dst_v, ones_v):
        c = lax.axis_index("c")
        s = lax.axis_index("s")
        pltpu.sync_copy(ones_hbm, ones_v)
        pltpu.sync_copy(zeros_hbm.at[pl.ds(s * RPT, RPT)],
                        acc.at[pl.ds(s * RPT, RPT)])
        plsc.subcore_barrier()

        def body(j, carry):
            pltpu.sync_copy(ones_v, acc.at[dst_v.at[j]], add=True)
            return carry

        @pl.when(c == 0)
        def _():
            pltpu.sync_copy(dst0_hbm.at[s], dst_v.at[pl.ds(0, CH0)])
            lax.fori_loop(0, CH0, body, 0)

        @pl.when(c == 1)
        def _():
            pltpu.sync_copy(dst1_hbm.at[s], dst_v.at[pl.ds(0, CH1)])
            lax.fori_loop(0, CH1, body, 0)

        plsc.subcore_barrier()
        pltpu.sync_copy(acc.at[pl.ds(s * RPT, RPT)],
                        out_hbm.at[c, pl.ds(s * RPT, RPT)])

    return k(dstp[0], dstp[1], ones128, zerosD)


def _sc_scatter(g, srcp, dstp, zerosD):
    @functools.partial(
        pl.kernel,
        mesh=_sc_mesh(),
        out_type=jax.ShapeDtypeStruct((2, NP_ROWS, D), jnp.float32),
        scratch_types=[
            pltpu.VMEM_SHARED((NP_ROWS, D), jnp.float32),
            pltpu.VMEM((CHMAX, CHUNK), jnp.int32),
            pltpu.VMEM((CHMAX, CHUNK), jnp.int32),
            pltpu.VMEM((CHUNK, D), jnp.float32),
            pltpu.SemaphoreType.DMA,
        ],
    )
    def k(g_hbm, src0_hbm, dst0_hbm, src1_hbm, dst1_hbm, zeros_hbm, out_hbm,
          acc, src_v, dst_v, rows_v, sem):
        c = lax.axis_index("c")
        s = lax.axis_index("s")
        pltpu.sync_copy(zeros_hbm.at[pl.ds(s * RPT, RPT)],
                        acc.at[pl.ds(s * RPT, RPT)])
        plsc.subcore_barrier()

        def body(j, carry):
            pltpu.async_copy(g_hbm.at[src_v.at[j]], rows_v, sem).wait()
            pltpu.sync_copy(rows_v, acc.at[dst_v.at[j]], add=True)
            return carry

        @pl.when(c == 0)
        def _():
            pltpu.sync_copy(src0_hbm.at[s], src_v.at[pl.ds(0, CH0)])
            pltpu.sync_copy(dst0_hbm.at[s], dst_v.at[pl.ds(0, CH0)])
            lax.fori_loop(0, CH0, body, 0)

        @pl.when(c == 1)
        def _():
            pltpu.sync_copy(src1_hbm.at[s], src_v.at[pl.ds(0, CH1)])
            pltpu.sync_copy(dst1_hbm.at[s], dst_v.at[pl.ds(0, CH1)])
            lax.fori_loop(0, CH1, body, 0)

        plsc.subcore_barrier()
        pltpu.sync_copy(acc.at[pl.ds(s * RPT, RPT)],
                        out_hbm.at[c, pl.ds(s * RPT, RPT)])

    return k(g, srcp[0], dstp[0], srcp[1], dstp[1], zerosD)


def _tc_pre(x, W0, i0, i1):
    def body(x_ref, w_ref, i0_ref, i1_ref, dinv_ref, g_ref):
        deg = 1.0 + i0_ref[:, :1] + i1_ref[:, :1]
        dinv = jnp.broadcast_to(lax.rsqrt(deg), (BN, D))
        dinv_ref[...] = dinv
        g_ref[...] = dinv * jnp.dot(x_ref[...], w_ref[...],
                                    preferred_element_type=jnp.float32)

    return pl.pallas_call(
        body,
        grid=(N // BN,),
        in_specs=[
            pl.BlockSpec((BN, D), lambda i: (i, 0)),
            pl.BlockSpec((D, D), lambda i: (0, 0)),
            pl.BlockSpec((BN, D), lambda i: (i, 0)),
            pl.BlockSpec((BN, D), lambda i: (i, 0)),
        ],
        out_specs=[pl.BlockSpec((BN, D), lambda i: (i, 0))] * 2,
        out_shape=[jax.ShapeDtypeStruct((N, D), jnp.float32)] * 2,
    )(x, W0, i0, i1)


def _tc_layer(S, g, dinv, b, Wn):
    def body(s0_ref, s1_ref, g_ref, dinv_ref, b_ref, w_ref, h_ref, gn_ref):
        dinv = dinv_ref[...]
        agg = dinv * (s0_ref[...] + s1_ref[...] + g_ref[...]) + b_ref[...]
        h = jnp.maximum(agg, 0.0)
        h_ref[...] = h
        gn_ref[...] = dinv * jnp.dot(h, w_ref[...],
                                     preferred_element_type=jnp.float32)

    return pl.pallas_call(
        body,
        grid=(N // BN,),
        in_specs=[
            pl.BlockSpec((BN, D), lambda i: (i, 0)),
            pl.BlockSpec((BN, D), lambda i: (i, 0)),
            pl.BlockSpec((BN, D), lambda i: (i, 0)),
            pl.BlockSpec((BN, D), lambda i: (i, 0)),
            pl.BlockSpec((1, D), lambda i: (0, 0)),
            pl.BlockSpec((D, D), lambda i: (0, 0)),
        ],
        out_specs=[pl.BlockSpec((BN, D), lambda i: (i, 0))] * 2,
        out_shape=[jax.ShapeDtypeStruct((N, D), jnp.float32)] * 2,
    )(S[0, :N, :], S[1, :N, :], g, dinv, b, Wn)


def _tc_final(S, g, dinv, b, Wffn):
    def body(s0_ref, s1_ref, g_ref, dinv_ref, b_ref, w_ref,
             h_ref, f_ref, ls_ref, sm_ref):
        dinv = dinv_ref[...]
        agg = dinv * (s0_ref[...] + s1_ref[...] + g_ref[...]) + b_ref[...]
        h = jnp.maximum(agg, 0.0)
        h_ref[...] = h
        f = jnp.maximum(jnp.dot(h, w_ref[...],
                                preferred_element_type=jnp.float32), 0.0)
        m = jnp.max(f, axis=1, keepdims=True)
        lse = m + jnp.log(jnp.sum(jnp.exp(f - m), axis=1, keepdims=True))
        ls = f - lse
        f_ref[...] = f
        ls_ref[...] = ls
        sm_ref[...] = jnp.exp(ls)

    return pl.pallas_call(
        body,
        grid=(N // BN,),
        in_specs=[
            pl.BlockSpec((BN, D), lambda i: (i, 0)),
            pl.BlockSpec((BN, D), lambda i: (i, 0)),
            pl.BlockSpec((BN, D), lambda i: (i, 0)),
            pl.BlockSpec((BN, D), lambda i: (i, 0)),
            pl.BlockSpec((1, D), lambda i: (0, 0)),
            pl.BlockSpec((D, DOUT), lambda i: (0, 0)),
        ],
        out_specs=[pl.BlockSpec((BN, D), lambda i: (i, 0))]
        + [pl.BlockSpec((BN, DOUT), lambda i: (i, 0))] * 3,
        out_shape=[jax.ShapeDtypeStruct((N, D), jnp.float32)]
        + [jax.ShapeDtypeStruct((N, DOUT), jnp.float32)] * 3,
    )(S[0, :N, :], S[1, :N, :], g, dinv, b, Wffn)


def kernel(x, edge_index, batch, W0, b0, W1, b1, W2, b2, Wffn):
    src = edge_index[0]
    dst = edge_index[1]
    pad = EP - E
    E0 = 16 * CH0 * CHUNK
    src_flat = jnp.concatenate([src, jnp.zeros((pad,), jnp.int32)])
    dst_flat = jnp.concatenate([dst, jnp.full((pad,), N, jnp.int32)])
    srcp = (src_flat[:E0].reshape(16, CH0, CHUNK),
            src_flat[E0:].reshape(16, CH1, CHUNK))
    dstp = (dst_flat[:E0].reshape(16, CH0, CHUNK),
            dst_flat[E0:].reshape(16, CH1, CHUNK))
    ones128 = jnp.ones((CHUNK, D), jnp.float32)
    zerosD = jnp.zeros((NP_ROWS, D), jnp.float32)

    indeg = _sc_degree(dstp, ones128, zerosD)
    i0 = indeg[0, :N, :]
    i1 = indeg[1, :N, :]
    dinv, g0 = _tc_pre(x, W0, i0, i1)

    S0 = _sc_scatter(g0, srcp, dstp, zerosD)
    h1, g1 = _tc_layer(S0, g0, dinv, b0.reshape(1, D), W1)

    S1 = _sc_scatter(g1, srcp, dstp, zerosD)
    h2, g2 = _tc_layer(S1, g1, dinv, b1.reshape(1, D), W2)

    S2 = _sc_scatter(g2, srcp, dstp, zerosD)
    h3, ffn_out, log_soft, soft = _tc_final(S2, g2, dinv, b2.reshape(1, D), Wffn)

    return (h1, h2, h3, h3, ffn_out, log_soft, soft)
